# baseline (speedup 1.0000x reference)
import functools

import jax
import jax.numpy as jnp
from jax import lax
from jax.experimental import pallas as pl
from jax.experimental.pallas import tpu as pltpu
from jax.experimental.pallas import tpu_sc as plsc

_N = 10000
_E = 320000
_G = 64
_F = 128
_C = 40
_CP = 128

_NC, _NS = 2, 16
_NW = _NC * _NS
_CH = 64
_NCHUNK = 160
_EPW = _NCHUNK * _CH
_EPAD = _NW * _EPW
_NPAD = 10240
_ZCH = 16
_HCH = _NCHUNK // 2
_KPT = _NPAD // _NS // _ZCH


def _make_edge_agg(d):
  mesh = plsc.VectorSubcoreMesh(core_axis_name="c", subcore_axis_name="s")

  @functools.partial(
      pl.kernel,
      mesh=mesh,
      out_type=jax.ShapeDtypeStruct((_NC, _NPAD, d), jnp.float32),
      scratch_types=[
          pltpu.VMEM((2, _HCH, _CH), jnp.int32),
          pltpu.VMEM((2, _CH, d), jnp.float32),
          pltpu.VMEM((_ZCH, d), jnp.float32),
          pltpu.VMEM_SHARED((_NPAD, d), jnp.float32),
          pltpu.SemaphoreType.DMA,
          pltpu.SemaphoreType.DMA,
      ],
  )
  def agg(table_hbm, src_hbm, dst_hbm, out_hbm,
          idx_sl, msgs_v, zbuf_v, acc_sh, sem0, sem1):
    c = lax.axis_index("c")
    s = lax.axis_index("s")
    wid = s * _NC + c

    def _zrow(i, carry):
      for j in range(d // 16):
        zbuf_v[i, pl.ds(j * 16, 16)] = jnp.zeros((16,), jnp.float32)
      return carry

    lax.fori_loop(0, _ZCH, _zrow, 0)
    for k in range(_KPT):
      pltpu.sync_copy(zbuf_v, acc_sh.at[pl.ds((s * _KPT + k) * _ZCH, _ZCH)])
    plsc.subcore_barrier()

    for half in range(2):
      pltpu.sync_copy(src_hbm.at[wid, pl.ds(half * _HCH, _HCH)], idx_sl.at[0])
      pltpu.sync_copy(dst_hbm.at[wid, pl.ds(half * _HCH, _HCH)], idx_sl.at[1])
      pltpu.async_copy(table_hbm.at[idx_sl.at[0, 0]], msgs_v.at[0], sem0)

      def _pair(i, carry):
        j1 = 2 * i + 1
        j2 = (2 * i + 2) % _HCH
        pltpu.make_async_copy(
            table_hbm.at[idx_sl.at[0, 0]], msgs_v.at[0], sem0).wait()
        pltpu.async_copy(table_hbm.at[idx_sl.at[0, j1]], msgs_v.at[1], sem1)
        pltpu.sync_copy(msgs_v.at[0], acc_sh.at[idx_sl.at[1, 2 * i]], add=True)
        pltpu.make_async_copy(
            table_hbm.at[idx_sl.at[0, 0]], msgs_v.at[1], sem1).wait()
        pltpu.async_copy(table_hbm.at[idx_sl.at[0, j2]], msgs_v.at[0], sem0)
        pltpu.sync_copy(msgs_v.at[1], acc_sh.at[idx_sl.at[1, j1]], add=True)
        return carry

      lax.fori_loop(0, _HCH // 2, _pair, 0)
      pltpu.make_async_copy(
          table_hbm.at[idx_sl.at[0, 0]], msgs_v.at[0], sem0).wait()
    plsc.subcore_barrier()

    for k in range(_KPT):
      r0 = (s * _KPT + k) * _ZCH
      pltpu.sync_copy(acc_sh.at[pl.ds(r0, _ZCH)], zbuf_v)
      pltpu.sync_copy(zbuf_v, out_hbm.at[c, pl.ds(r0, _ZCH)])

  return agg


_agg_x = _make_edge_agg(_F)
_agg_q = _agg_x if _CP == _F else _make_edge_agg(_CP)


def _tc_transform(p, w1, w2p):
  def body(p0, p1, a, b, o):
    t = jnp.dot(p0[...] + p1[...], a[...], preferred_element_type=jnp.float32)
    t = jnp.maximum(t, 0.0)
    o[...] = jnp.dot(t, b[...], preferred_element_type=jnp.float32)

  blk = 1280
  return pl.pallas_call(
      body,
      grid=(_NPAD // blk,),
      in_specs=[
          pl.BlockSpec((None, blk, _F), lambda i: (0, i, 0)),
          pl.BlockSpec((None, blk, _F), lambda i: (1, i, 0)),
          pl.BlockSpec((_F, _F), lambda i: (0, 0)),
          pl.BlockSpec((_F, _CP), lambda i: (0, 0)),
      ],
      out_specs=pl.BlockSpec((blk, _CP), lambda i: (i, 0)),
      out_shape=jax.ShapeDtypeStruct((_NPAD, _CP), jnp.float32),
  )(p, p, w1, w2p)


def _tc_pool(p2, batch2d):
  def body(p0, p1, b, o):
    h2 = p0[...] + p1[...]
    gids = lax.broadcasted_iota(jnp.int32, (_NPAD, _G), 1)
    onehot = jnp.where(b[...] == gids, 1.0, 0.0).astype(jnp.float32)
    sums = lax.dot_general(onehot, h2, (((0,), (0,)), ((), ())),
                           preferred_element_type=jnp.float32)
    counts = jnp.maximum(jnp.sum(onehot, axis=0), 1.0)
    o[...] = sums / counts[:, None]

  return pl.pallas_call(
      body,
      grid=(1,),
      in_specs=[
          pl.BlockSpec((None, _NPAD, _CP), lambda i: (0, 0, 0)),
          pl.BlockSpec((None, _NPAD, _CP), lambda i: (1, 0, 0)),
          pl.BlockSpec((_NPAD, 1), lambda i: (0, 0)),
      ],
      out_specs=pl.BlockSpec((_G, _CP), lambda i: (0, 0)),
      out_shape=jax.ShapeDtypeStruct((_G, _CP), jnp.float32),
  )(p2, p2, batch2d)


def kernel(x, edge_index, batch, W1, W2):
  src = edge_index[0].astype(jnp.int32)
  dst = edge_index[1].astype(jnp.int32)
  pad = _EPAD - _E
  src_p = jnp.concatenate([src, jnp.zeros((pad,), jnp.int32)])
  dst_p = jnp.concatenate([dst, jnp.full((pad,), _N, jnp.int32)])
  src_p = src_p.reshape(_NW, _NCHUNK, _CH)
  dst_p = dst_p.reshape(_NW, _NCHUNK, _CH)

  p1 = _agg_x(x, src_p, dst_p)
  w2p = jnp.pad(W2, ((0, 0), (0, _CP - _C)))
  q = _tc_transform(p1, W1, w2p)
  p2 = _agg_q(q, src_p, dst_p)

  bpad = jnp.concatenate(
      [batch.astype(jnp.int32), jnp.full((_NPAD - _N,), _G, jnp.int32)])
  out = _tc_pool(p2, bpad.reshape(_NPAD, 1))
  return out[:, :_C]


# speedup vs baseline: 1.1151x; 1.1151x over previous
import functools

import jax
import jax.numpy as jnp
from jax import lax
from jax.experimental import pallas as pl
from jax.experimental.pallas import tpu as pltpu
from jax.experimental.pallas import tpu_sc as plsc

_N = 10000
_E = 320000
_G = 64
_F = 128
_C = 40
_CP = 128

_NC, _NS = 2, 16
_NW = _NC * _NS
_CH = 128
_NCHUNK = 80
_EPW = _NCHUNK * _CH
_EPAD = _NW * _EPW
_NPAD = 10240
_ZCH = 16
_HCH = 16
_KPT = _NPAD // _NS // _ZCH


def _make_edge_agg(d):
  mesh = plsc.VectorSubcoreMesh(core_axis_name="c", subcore_axis_name="s")

  @functools.partial(
      pl.kernel,
      mesh=mesh,
      out_type=jax.ShapeDtypeStruct((_NC, _NPAD, d), jnp.float32),
      scratch_types=[
          pltpu.VMEM((2, _HCH, _CH), jnp.int32),
          pltpu.VMEM((2, _CH, d), jnp.float32),
          pltpu.VMEM((_ZCH, d), jnp.float32),
          pltpu.VMEM_SHARED((_NPAD, d), jnp.float32),
          pltpu.SemaphoreType.DMA,
          pltpu.SemaphoreType.DMA,
          pltpu.SemaphoreType.DMA,
          pltpu.SemaphoreType.DMA,
      ],
  )
  def agg(table_hbm, src_hbm, dst_hbm, out_hbm,
          idx_sl, msgs_v, zbuf_v, acc_sh, g0, g1, s0, s1):
    c = lax.axis_index("c")
    s = lax.axis_index("s")
    wid = s * _NC + c

    def _zrow(i, carry):
      for j in range(d // 16):
        zbuf_v[i, pl.ds(j * 16, 16)] = jnp.zeros((16,), jnp.float32)
      return carry

    lax.fori_loop(0, _ZCH, _zrow, 0)
    for k in range(_KPT):
      pltpu.sync_copy(zbuf_v, acc_sh.at[pl.ds((s * _KPT + k) * _ZCH, _ZCH)])
    plsc.subcore_barrier()

    def _wait_gather(sem, b):
      pltpu.make_async_copy(
          table_hbm.at[idx_sl.at[0, 0]], msgs_v.at[b], sem).wait()

    def _wait_scatter(sem, b):
      pltpu.make_async_copy(
          table_hbm.at[idx_sl.at[0, 0]], msgs_v.at[b], sem).wait()

    npair = _HCH // 2
    for sl in range(_NCHUNK // _HCH):
      pltpu.sync_copy(src_hbm.at[wid, pl.ds(sl * _HCH, _HCH)], idx_sl.at[0])
      pltpu.sync_copy(dst_hbm.at[wid, pl.ds(sl * _HCH, _HCH)], idx_sl.at[1])
      pltpu.async_copy(table_hbm.at[idx_sl.at[0, 0]], msgs_v.at[0], g0)

      def _pair(i, carry):
        j0, j1 = 2 * i, 2 * i + 1
        _wait_gather(g0, 0)

        @pl.when(i > 0)
        def _():
          _wait_scatter(s1, 1)

        pltpu.async_copy(table_hbm.at[idx_sl.at[0, j1]], msgs_v.at[1], g1)
        pltpu.async_copy(msgs_v.at[0], acc_sh.at[idx_sl.at[1, j0]], s0,
                         add=True)
        _wait_gather(g1, 1)
        _wait_scatter(s0, 0)

        @pl.when(i < npair - 1)
        def _():
          pltpu.async_copy(table_hbm.at[idx_sl.at[0, j0 + 2]], msgs_v.at[0],
                           g0)

        pltpu.async_copy(msgs_v.at[1], acc_sh.at[idx_sl.at[1, j1]], s1,
                         add=True)
        return carry

      lax.fori_loop(0, npair, _pair, 0)
      _wait_scatter(s1, 1)
    plsc.subcore_barrier()

    for k in range(_KPT):
      r0 = (s * _KPT + k) * _ZCH
      pltpu.sync_copy(acc_sh.at[pl.ds(r0, _ZCH)], zbuf_v)
      pltpu.sync_copy(zbuf_v, out_hbm.at[c, pl.ds(r0, _ZCH)])

  return agg


_agg_x = _make_edge_agg(_F)
_agg_q = _agg_x if _CP == _F else _make_edge_agg(_CP)


def _tc_transform(p, w1, w2p):
  def body(p0, p1, a, b, o):
    t = jnp.dot(p0[...] + p1[...], a[...], preferred_element_type=jnp.float32)
    t = jnp.maximum(t, 0.0)
    o[...] = jnp.dot(t, b[...], preferred_element_type=jnp.float32)

  blk = 1280
  return pl.pallas_call(
      body,
      grid=(_NPAD // blk,),
      in_specs=[
          pl.BlockSpec((None, blk, _F), lambda i: (0, i, 0)),
          pl.BlockSpec((None, blk, _F), lambda i: (1, i, 0)),
          pl.BlockSpec((_F, _F), lambda i: (0, 0)),
          pl.BlockSpec((_F, _CP), lambda i: (0, 0)),
      ],
      out_specs=pl.BlockSpec((blk, _CP), lambda i: (i, 0)),
      out_shape=jax.ShapeDtypeStruct((_NPAD, _CP), jnp.float32),
  )(p, p, w1, w2p)


def _tc_pool(p2, batch2d):
  def body(p0, p1, b, o):
    h2 = p0[...] + p1[...]
    gids = lax.broadcasted_iota(jnp.int32, (_NPAD, _G), 1)
    onehot = jnp.where(b[...] == gids, 1.0, 0.0).astype(jnp.float32)
    sums = lax.dot_general(onehot, h2, (((0,), (0,)), ((), ())),
                           preferred_element_type=jnp.float32)
    counts = jnp.maximum(jnp.sum(onehot, axis=0), 1.0)
    o[...] = sums / counts[:, None]

  return pl.pallas_call(
      body,
      grid=(1,),
      in_specs=[
          pl.BlockSpec((None, _NPAD, _CP), lambda i: (0, 0, 0)),
          pl.BlockSpec((None, _NPAD, _CP), lambda i: (1, 0, 0)),
          pl.BlockSpec((_NPAD, 1), lambda i: (0, 0)),
      ],
      out_specs=pl.BlockSpec((_G, _CP), lambda i: (0, 0)),
      out_shape=jax.ShapeDtypeStruct((_G, _CP), jnp.float32),
  )(p2, p2, batch2d)


def kernel(x, edge_index, batch, W1, W2):
  src = edge_index[0].astype(jnp.int32)
  dst = edge_index[1].astype(jnp.int32)
  pad = _EPAD - _E
  src_p = jnp.concatenate([src, jnp.zeros((pad,), jnp.int32)])
  dst_p = jnp.concatenate([dst, jnp.full((pad,), _N, jnp.int32)])
  src_p = src_p.reshape(_NW, _NCHUNK, _CH)
  dst_p = dst_p.reshape(_NW, _NCHUNK, _CH)

  p1 = _agg_x(x, src_p, dst_p)
  w2p = jnp.pad(W2, ((0, 0), (0, _CP - _C)))
  q = _tc_transform(p1, W1, w2p)
  p2 = _agg_q(q, src_p, dst_p)

  bpad = jnp.concatenate(
      [batch.astype(jnp.int32), jnp.full((_NPAD - _N,), _G, jnp.int32)])
  out = _tc_pool(p2, bpad.reshape(_NPAD, 1))
  return out[:, :_C]


# speedup vs baseline: 1.1152x; 1.0001x over previous
import functools

import jax
import jax.numpy as jnp
from jax import lax
from jax.experimental import pallas as pl
from jax.experimental.pallas import tpu as pltpu
from jax.experimental.pallas import tpu_sc as plsc

_N = 10000
_E = 320000
_G = 64
_F = 128
_C = 40
_CP = 128

_NC, _NS = 2, 16
_NW = _NC * _NS
_CH = 128
_NCHUNK = 80
_EPW = _NCHUNK * _CH
_EPAD = _NW * _EPW
_NPAD = 10240
_ZCH = 16
_HCH = 16
_KPT = _NPAD // _NS // _ZCH


def _make_edge_agg(d):
  mesh = plsc.VectorSubcoreMesh(core_axis_name="c", subcore_axis_name="s")

  @functools.partial(
      pl.kernel,
      mesh=mesh,
      out_type=jax.ShapeDtypeStruct((_NC, _NPAD, d), jnp.float32),
      scratch_types=[
          pltpu.VMEM((2, _HCH, _CH), jnp.int32),
          pltpu.VMEM((2, _CH, d), jnp.float32),
          pltpu.VMEM((_ZCH, d), jnp.float32),
          pltpu.VMEM_SHARED((_NPAD, d), jnp.float32),
          pltpu.SemaphoreType.DMA,
          pltpu.SemaphoreType.DMA,
          pltpu.SemaphoreType.DMA,
          pltpu.SemaphoreType.DMA,
      ],
  )
  def agg(table_hbm, src_hbm, dst_hbm, out_hbm,
          idx_sl, msgs_v, zbuf_v, acc_sh, g0, g1, s0, s1):
    c = lax.axis_index("c")
    s = lax.axis_index("s")
    wid = s * _NC + c

    def _zrow(i, carry):
      for j in range(d // 16):
        zbuf_v[i, pl.ds(j * 16, 16)] = jnp.zeros((16,), jnp.float32)
      return carry

    lax.fori_loop(0, _ZCH, _zrow, 0)
    for k in range(_KPT):
      pltpu.sync_copy(zbuf_v, acc_sh.at[pl.ds((s * _KPT + k) * _ZCH, _ZCH)])
    plsc.subcore_barrier()

    def _wait_gather(sem, b):
      pltpu.make_async_copy(
          table_hbm.at[idx_sl.at[0, 0]], msgs_v.at[b], sem).wait()

    def _wait_scatter(sem, b):
      pltpu.make_async_copy(
          table_hbm.at[idx_sl.at[0, 0]], msgs_v.at[b], sem).wait()

    npair = _HCH // 2
    for sl in range(_NCHUNK // _HCH):
      pltpu.sync_copy(src_hbm.at[wid, pl.ds(sl * _HCH, _HCH)], idx_sl.at[0])
      pltpu.sync_copy(dst_hbm.at[wid, pl.ds(sl * _HCH, _HCH)], idx_sl.at[1])
      pltpu.async_copy(table_hbm.at[idx_sl.at[0, 0]], msgs_v.at[0], g0)

      def _pair(i, carry):
        j0, j1 = 2 * i, 2 * i + 1
        _wait_gather(g0, 0)

        @pl.when(i > 0)
        def _():
          _wait_scatter(s1, 1)

        pltpu.async_copy(table_hbm.at[idx_sl.at[0, j1]], msgs_v.at[1], g1)
        pltpu.async_copy(msgs_v.at[0], acc_sh.at[idx_sl.at[1, j0]], s0,
                         add=True)
        _wait_gather(g1, 1)
        _wait_scatter(s0, 0)

        @pl.when(i < npair - 1)
        def _():
          pltpu.async_copy(table_hbm.at[idx_sl.at[0, j0 + 2]], msgs_v.at[0],
                           g0)

        pltpu.async_copy(msgs_v.at[1], acc_sh.at[idx_sl.at[1, j1]], s1,
                         add=True)
        return carry

      lax.fori_loop(0, npair, _pair, 0)
      _wait_scatter(s1, 1)
    plsc.subcore_barrier()

    for k in range(_KPT):
      r0 = (s * _KPT + k) * _ZCH
      pltpu.sync_copy(acc_sh.at[pl.ds(r0, _ZCH)], zbuf_v)
      pltpu.sync_copy(zbuf_v, out_hbm.at[c, pl.ds(r0, _ZCH)])

  return agg


_agg_x = _make_edge_agg(_F)
_agg_q = _agg_x if _CP == _F else _make_edge_agg(_CP)


def _tc_transform(p, w1, w2p):
  def body(p0, p1, a, b, o):
    t = jnp.dot(p0[...] + p1[...], a[...], preferred_element_type=jnp.float32)
    t = jnp.maximum(t, 0.0)
    o[...] = jnp.dot(t, b[...], preferred_element_type=jnp.float32)

  blk = 1280
  return pl.pallas_call(
      body,
      grid=(_NPAD // blk,),
      in_specs=[
          pl.BlockSpec((None, blk, _F), lambda i: (0, i, 0)),
          pl.BlockSpec((None, blk, _F), lambda i: (1, i, 0)),
          pl.BlockSpec((_F, _F), lambda i: (0, 0)),
          pl.BlockSpec((_F, _CP), lambda i: (0, 0)),
      ],
      out_specs=pl.BlockSpec((blk, _CP), lambda i: (i, 0)),
      out_shape=jax.ShapeDtypeStruct((_NPAD, _CP), jnp.float32),
  )(p, p, w1, w2p)


def _tc_pool(p2, batch2d):
  def body(p0, p1, b, o):
    h2 = p0[...] + p1[...]
    gids = lax.broadcasted_iota(jnp.int32, (_NPAD, _G), 1)
    onehot = jnp.where(b[...] == gids, 1.0, 0.0).astype(jnp.float32)
    sums = lax.dot_general(onehot, h2, (((0,), (0,)), ((), ())),
                           preferred_element_type=jnp.float32)
    counts = jnp.maximum(jnp.sum(onehot, axis=0), 1.0)
    o[...] = sums / counts[:, None]

  return pl.pallas_call(
      body,
      grid=(1,),
      in_specs=[
          pl.BlockSpec((None, _NPAD, _CP), lambda i: (0, 0, 0)),
          pl.BlockSpec((None, _NPAD, _CP), lambda i: (1, 0, 0)),
          pl.BlockSpec((_NPAD, 1), lambda i: (0, 0)),
      ],
      out_specs=pl.BlockSpec((_G, _CP), lambda i: (0, 0)),
      out_shape=jax.ShapeDtypeStruct((_G, _CP), jnp.float32),
  )(p2, p2, batch2d)


def kernel(x, edge_index, batch, W1, W2):
  src = edge_index[0].astype(jnp.int32)
  dst = edge_index[1].astype(jnp.int32)
  pad = _EPAD - _E
  src_p = jnp.concatenate([src, jnp.zeros((pad,), jnp.int32)])
  dummy = _N + (jnp.arange(pad, dtype=jnp.int32) % (_NPAD - _N))
  dst_p = jnp.concatenate([dst, dummy])
  src_p = src_p.reshape(_NW, _NCHUNK, _CH)
  dst_p = dst_p.reshape(_NW, _NCHUNK, _CH)

  p1 = _agg_x(x, src_p, dst_p)
  w2p = jnp.pad(W2, ((0, 0), (0, _CP - _C)))
  q = _tc_transform(p1, W1, w2p)
  p2 = _agg_q(q, src_p, dst_p)

  bpad = jnp.concatenate(
      [batch.astype(jnp.int32), jnp.full((_NPAD - _N,), _G, jnp.int32)])
  out = _tc_pool(p2, bpad.reshape(_NPAD, 1))
  return out[:, :_C]


# speedup vs baseline: 3.0410x; 2.7268x over previous
import functools

import jax
import jax.numpy as jnp
from jax import lax
from jax.experimental import pallas as pl
from jax.experimental.pallas import tpu as pltpu
from jax.experimental.pallas import tpu_sc as plsc

_N = 10000
_E = 320000
_G = 64
_F = 128
_C = 40
_CP = 128

_NC, _NS = 2, 16
_NW = _NC * _NS
_CH = 128
_NCHUNK = 80
_EPW = _NCHUNK * _CH
_EPAD = _NW * _EPW
_NPAD = 10240
_ZCH = 16
_HCH = 16
_KPT = _NPAD // _NS // _ZCH


def _make_edge_agg(d):
  mesh = plsc.VectorSubcoreMesh(core_axis_name="c", subcore_axis_name="s")

  @functools.partial(
      pl.kernel,
      mesh=mesh,
      out_type=jax.ShapeDtypeStruct((_NC, _NPAD, d), jnp.float32),
      scratch_types=[
          pltpu.VMEM((2, _HCH, _CH), jnp.int32),
          pltpu.VMEM((2, _CH, d), jnp.float32),
          pltpu.VMEM((_ZCH, d), jnp.float32),
          pltpu.VMEM_SHARED((_NPAD, d), jnp.float32),
          pltpu.SemaphoreType.DMA,
          pltpu.SemaphoreType.DMA,
          pltpu.SemaphoreType.DMA,
          pltpu.SemaphoreType.DMA,
      ],
  )
  def agg(table_hbm, src_hbm, dst_hbm, out_hbm,
          idx_sl, msgs_v, zbuf_v, acc_sh, g0, g1, s0, s1):
    c = lax.axis_index("c")
    s = lax.axis_index("s")
    wid = s * _NC + c

    def _zrow(i, carry):
      for j in range(d // 16):
        zbuf_v[i, pl.ds(j * 16, 16)] = jnp.zeros((16,), jnp.float32)
      return carry

    lax.fori_loop(0, _ZCH, _zrow, 0)
    for k in range(_KPT):
      pltpu.sync_copy(zbuf_v, acc_sh.at[pl.ds((s * _KPT + k) * _ZCH, _ZCH)])
    plsc.subcore_barrier()

    def _wait_gather(sem, b):
      pltpu.make_async_copy(
          table_hbm.at[idx_sl.at[0, 0]], msgs_v.at[b], sem).wait()

    def _wait_scatter(sem, b):
      pltpu.make_async_copy(
          table_hbm.at[idx_sl.at[0, 0]], msgs_v.at[b], sem).wait()

    npair = _HCH // 2
    for sl in range(_NCHUNK // _HCH):
      pltpu.sync_copy(src_hbm.at[wid, pl.ds(sl * _HCH, _HCH)], idx_sl.at[0])
      pltpu.sync_copy(dst_hbm.at[wid, pl.ds(sl * _HCH, _HCH)], idx_sl.at[1])
      pltpu.async_copy(table_hbm.at[idx_sl.at[0, 0]], msgs_v.at[0], g0)

      def _pair(i, carry):
        j0, j1 = 2 * i, 2 * i + 1
        _wait_gather(g0, 0)

        @pl.when(i > 0)
        def _():
          _wait_scatter(s1, 1)

        pltpu.async_copy(table_hbm.at[idx_sl.at[0, j1]], msgs_v.at[1], g1)
        pltpu.async_copy(msgs_v.at[0], acc_sh.at[idx_sl.at[1, j0]], s0,
                         add=True)
        _wait_gather(g1, 1)
        _wait_scatter(s0, 0)

        @pl.when(i < npair - 1)
        def _():
          pltpu.async_copy(table_hbm.at[idx_sl.at[0, j0 + 2]], msgs_v.at[0],
                           g0)

        pltpu.async_copy(msgs_v.at[1], acc_sh.at[idx_sl.at[1, j1]], s1,
                         add=True)
        return carry

      lax.fori_loop(0, npair, _pair, 0)
      _wait_scatter(s1, 1)
    plsc.subcore_barrier()

    for k in range(_KPT):
      r0 = (s * _KPT + k) * _ZCH
      pltpu.sync_copy(acc_sh.at[pl.ds(r0, _ZCH)], zbuf_v)
      pltpu.sync_copy(zbuf_v, out_hbm.at[c, pl.ds(r0, _ZCH)])

  return agg


_agg_x = _make_edge_agg(_F)
_agg_q = _agg_x if _CP == _F else _make_edge_agg(_CP)


def _tc_transform(p, w1, w2p):
  def body(p0, p1, a, b, o):
    t = jnp.dot(p0[...] + p1[...], a[...], preferred_element_type=jnp.float32)
    t = jnp.maximum(t, 0.0)
    o[...] = jnp.dot(t, b[...], preferred_element_type=jnp.float32)

  blk = 1280
  return pl.pallas_call(
      body,
      grid=(_NPAD // blk,),
      in_specs=[
          pl.BlockSpec((None, blk, _F), lambda i: (0, i, 0)),
          pl.BlockSpec((None, blk, _F), lambda i: (1, i, 0)),
          pl.BlockSpec((_F, _F), lambda i: (0, 0)),
          pl.BlockSpec((_F, _CP), lambda i: (0, 0)),
      ],
      out_specs=pl.BlockSpec((blk, _CP), lambda i: (i, 0)),
      out_shape=jax.ShapeDtypeStruct((_NPAD, _CP), jnp.float32),
  )(p, p, w1, w2p)


def _tc_pool(p2, batch2d):
  def body(p0, p1, b, o):
    h2 = p0[...] + p1[...]
    gids = lax.broadcasted_iota(jnp.int32, (_NPAD, _G), 1)
    onehot = jnp.where(b[...] == gids, 1.0, 0.0).astype(jnp.float32)
    sums = lax.dot_general(onehot, h2, (((0,), (0,)), ((), ())),
                           preferred_element_type=jnp.float32)
    counts = jnp.maximum(jnp.sum(onehot, axis=0), 1.0)
    o[...] = sums / counts[:, None]

  return pl.pallas_call(
      body,
      grid=(1,),
      in_specs=[
          pl.BlockSpec((None, _NPAD, _CP), lambda i: (0, 0, 0)),
          pl.BlockSpec((None, _NPAD, _CP), lambda i: (1, 0, 0)),
          pl.BlockSpec((_NPAD, 1), lambda i: (0, 0)),
      ],
      out_specs=pl.BlockSpec((_G, _CP), lambda i: (0, 0)),
      out_shape=jax.ShapeDtypeStruct((_G, _CP), jnp.float32),
  )(p2, p2, batch2d)


def kernel(x, edge_index, batch, W1, W2):
  src = edge_index[0].astype(jnp.int32)
  dst = edge_index[1].astype(jnp.int32)
  pad = _EPAD - _E
  ar = jnp.arange(pad, dtype=jnp.int32)
  src_p = jnp.concatenate([src, ar % _N])
  dst_p = jnp.concatenate([dst, _N + (ar % (_NPAD - _N))])
  src_p = src_p.reshape(_NW, _NCHUNK, _CH)
  dst_p = dst_p.reshape(_NW, _NCHUNK, _CH)

  p1 = _agg_x(x, src_p, dst_p)
  w2p = jnp.pad(W2, ((0, 0), (0, _CP - _C)))
  q = _tc_transform(p1, W1, w2p)
  p2 = _agg_q(q, src_p, dst_p)

  bpad = jnp.concatenate(
      [batch.astype(jnp.int32), jnp.full((_NPAD - _N,), _G, jnp.int32)])
  out = _tc_pool(p2, bpad.reshape(_NPAD, 1))
  return out[:, :_C]


# speedup vs baseline: 3.3145x; 1.0899x over previous
import functools

import jax
import jax.numpy as jnp
from jax import lax
from jax.experimental import pallas as pl
from jax.experimental.pallas import tpu as pltpu
from jax.experimental.pallas import tpu_sc as plsc

_N = 10000
_E = 320000
_G = 64
_F = 128
_C = 40
_CP = 64

_NC, _NS = 2, 16
_NW = _NC * _NS
_CH = 128
_NCHUNK = 80
_EPW = _NCHUNK * _CH
_EPAD = _NW * _EPW
_NPAD = 10240
_ZCH = 16
_HCH = 16
_KPT = _NPAD // _NS // _ZCH


def _make_edge_agg(d, tc_tiling=True):
  mesh = plsc.VectorSubcoreMesh(core_axis_name="c", subcore_axis_name="s")
  extra = {}
  if not tc_tiling:
    extra["compiler_params"] = pltpu.CompilerParams(use_tc_tiling_on_sc=False)

  @functools.partial(
      pl.kernel,
      mesh=mesh,
      **extra,
      out_type=jax.ShapeDtypeStruct((_NC, _NPAD, d), jnp.float32),
      scratch_types=[
          pltpu.VMEM((2, _HCH, _CH), jnp.int32),
          pltpu.VMEM((2, _CH, d), jnp.float32),
          pltpu.VMEM((_ZCH, d), jnp.float32),
          pltpu.VMEM_SHARED((_NPAD, d), jnp.float32),
          pltpu.SemaphoreType.DMA,
          pltpu.SemaphoreType.DMA,
          pltpu.SemaphoreType.DMA,
          pltpu.SemaphoreType.DMA,
      ],
  )
  def agg(table_hbm, src_hbm, dst_hbm, out_hbm,
          idx_sl, msgs_v, zbuf_v, acc_sh, g0, g1, s0, s1):
    c = lax.axis_index("c")
    s = lax.axis_index("s")
    wid = s * _NC + c

    def _zrow(i, carry):
      for j in range(d // 16):
        zbuf_v[i, pl.ds(j * 16, 16)] = jnp.zeros((16,), jnp.float32)
      return carry

    lax.fori_loop(0, _ZCH, _zrow, 0)
    for k in range(_KPT):
      pltpu.sync_copy(zbuf_v, acc_sh.at[pl.ds((s * _KPT + k) * _ZCH, _ZCH)])
    plsc.subcore_barrier()

    def _wait_gather(sem, b):
      pltpu.make_async_copy(
          table_hbm.at[idx_sl.at[0, 0]], msgs_v.at[b], sem).wait()

    def _wait_scatter(sem, b):
      pltpu.make_async_copy(
          table_hbm.at[idx_sl.at[0, 0]], msgs_v.at[b], sem).wait()

    npair = _HCH // 2
    for sl in range(_NCHUNK // _HCH):
      pltpu.sync_copy(src_hbm.at[wid, pl.ds(sl * _HCH, _HCH)], idx_sl.at[0])
      pltpu.sync_copy(dst_hbm.at[wid, pl.ds(sl * _HCH, _HCH)], idx_sl.at[1])
      pltpu.async_copy(table_hbm.at[idx_sl.at[0, 0]], msgs_v.at[0], g0)

      def _pair(i, carry):
        j0, j1 = 2 * i, 2 * i + 1
        _wait_gather(g0, 0)

        @pl.when(i > 0)
        def _():
          _wait_scatter(s1, 1)

        pltpu.async_copy(table_hbm.at[idx_sl.at[0, j1]], msgs_v.at[1], g1)
        pltpu.async_copy(msgs_v.at[0], acc_sh.at[idx_sl.at[1, j0]], s0,
                         add=True)
        _wait_gather(g1, 1)
        _wait_scatter(s0, 0)

        @pl.when(i < npair - 1)
        def _():
          pltpu.async_copy(table_hbm.at[idx_sl.at[0, j0 + 2]], msgs_v.at[0],
                           g0)

        pltpu.async_copy(msgs_v.at[1], acc_sh.at[idx_sl.at[1, j1]], s1,
                         add=True)
        return carry

      lax.fori_loop(0, npair, _pair, 0)
      _wait_scatter(s1, 1)
    plsc.subcore_barrier()

    for k in range(_KPT):
      r0 = (s * _KPT + k) * _ZCH
      pltpu.sync_copy(acc_sh.at[pl.ds(r0, _ZCH)], zbuf_v)
      pltpu.sync_copy(zbuf_v, out_hbm.at[c, pl.ds(r0, _ZCH)])

  return agg


_agg_x = _make_edge_agg(_F)
_agg_q = _agg_x if _CP == _F else _make_edge_agg(_CP, tc_tiling=False)


def _tc_transform(p, w1, w2p):
  def body(p0, p1, a, b, o):
    t = jnp.dot(p0[...] + p1[...], a[...], preferred_element_type=jnp.float32)
    t = jnp.maximum(t, 0.0)
    o[...] = jnp.dot(t, b[...], preferred_element_type=jnp.float32)

  blk = 1280
  return pl.pallas_call(
      body,
      grid=(_NPAD // blk,),
      in_specs=[
          pl.BlockSpec((None, blk, _F), lambda i: (0, i, 0)),
          pl.BlockSpec((None, blk, _F), lambda i: (1, i, 0)),
          pl.BlockSpec((_F, _F), lambda i: (0, 0)),
          pl.BlockSpec((_F, _CP), lambda i: (0, 0)),
      ],
      out_specs=pl.BlockSpec((blk, _CP), lambda i: (i, 0)),
      out_shape=jax.ShapeDtypeStruct((_NPAD, _CP), jnp.float32),
  )(p, p, w1, w2p)


def _tc_pool(p2, batch2d):
  def body(p0, p1, b, o):
    h2 = p0[...] + p1[...]
    gids = lax.broadcasted_iota(jnp.int32, (_NPAD, _G), 1)
    onehot = jnp.where(b[...] == gids, 1.0, 0.0).astype(jnp.float32)
    sums = lax.dot_general(onehot, h2, (((0,), (0,)), ((), ())),
                           preferred_element_type=jnp.float32)
    counts = jnp.maximum(jnp.sum(onehot, axis=0), 1.0)
    o[...] = sums / counts[:, None]

  return pl.pallas_call(
      body,
      grid=(1,),
      in_specs=[
          pl.BlockSpec((None, _NPAD, _CP), lambda i: (0, 0, 0)),
          pl.BlockSpec((None, _NPAD, _CP), lambda i: (1, 0, 0)),
          pl.BlockSpec((_NPAD, 1), lambda i: (0, 0)),
      ],
      out_specs=pl.BlockSpec((_G, _CP), lambda i: (0, 0)),
      out_shape=jax.ShapeDtypeStruct((_G, _CP), jnp.float32),
  )(p2, p2, batch2d)


def kernel(x, edge_index, batch, W1, W2):
  src = edge_index[0].astype(jnp.int32)
  dst = edge_index[1].astype(jnp.int32)
  pad = _EPAD - _E
  ar = jnp.arange(pad, dtype=jnp.int32)
  src_p = jnp.concatenate([src, ar % _N])
  dst_p = jnp.concatenate([dst, _N + (ar % (_NPAD - _N))])
  src_p = src_p.reshape(_NW, _NCHUNK, _CH)
  dst_p = dst_p.reshape(_NW, _NCHUNK, _CH)

  p1 = _agg_x(x, src_p, dst_p)
  w2p = jnp.pad(W2, ((0, 0), (0, _CP - _C)))
  q = _tc_transform(p1, W1, w2p)
  p2 = _agg_q(q, src_p, dst_p)

  bpad = jnp.concatenate(
      [batch.astype(jnp.int32), jnp.full((_NPAD - _N,), _G, jnp.int32)])
  out = _tc_pool(p2, bpad.reshape(_NPAD, 1))
  return out[:, :_C]


# speedup vs baseline: 3.9799x; 1.2007x over previous
import functools

import jax
import jax.numpy as jnp
from jax import lax
from jax.experimental import pallas as pl
from jax.experimental.pallas import tpu as pltpu
from jax.experimental.pallas import tpu_sc as plsc

_N = 10000
_E = 320000
_G = 64
_F = 128
_C = 40
_CP = 64

_NC, _NS = 2, 16
_NW = _NC * _NS
_EPW = 10240
_EPAD = _NW * _EPW
_NPAD = 10240
_ZCH = 16
_KPT = _NPAD // _NS // _ZCH


def _make_edge_agg(d, ch, sch, tc_tiling=True):
  nchunk = _EPW // ch
  nslab = nchunk // sch
  m = sch // 4
  mesh = plsc.VectorSubcoreMesh(core_axis_name="c", subcore_axis_name="s")
  extra = {}
  if not tc_tiling:
    extra["compiler_params"] = pltpu.CompilerParams(use_tc_tiling_on_sc=False)

  @functools.partial(
      pl.kernel,
      mesh=mesh,
      **extra,
      out_type=jax.ShapeDtypeStruct((_NC, _NPAD, d), jnp.float32),
      scratch_types=[
          pltpu.VMEM((2, sch, ch), jnp.int32),
          pltpu.VMEM((4, ch, d), jnp.float32),
          pltpu.VMEM((_ZCH, d), jnp.float32),
          pltpu.VMEM_SHARED((_NPAD, d), jnp.float32),
      ] + [pltpu.SemaphoreType.DMA] * 8,
  )
  def agg(table_hbm, src_hbm, dst_hbm, out_hbm,
          idx_sl, msgs_v, zbuf_v, acc_sh,
          g0, g1, g2, g3, s0, s1, s2, s3):
    gsem = (g0, g1, g2, g3)
    ssem = (s0, s1, s2, s3)
    c = lax.axis_index("c")
    s = lax.axis_index("s")
    wid = s * _NC + c

    def _zrow(i, carry):
      for j in range(d // 16):
        zbuf_v[i, pl.ds(j * 16, 16)] = jnp.zeros((16,), jnp.float32)
      return carry

    lax.fori_loop(0, _ZCH, _zrow, 0)
    for k in range(_KPT):
      pltpu.sync_copy(zbuf_v, acc_sh.at[pl.ds((s * _KPT + k) * _ZCH, _ZCH)])
    plsc.subcore_barrier()

    def _gather(j, b, sem):
      pltpu.async_copy(table_hbm.at[idx_sl.at[0, j]], msgs_v.at[b], sem)

    def _scatter(j, b, sem):
      pltpu.async_copy(msgs_v.at[b], acc_sh.at[idx_sl.at[1, j]], sem,
                       add=True)

    def _wait(sem, b):
      pltpu.make_async_copy(
          table_hbm.at[idx_sl.at[0, 0]], msgs_v.at[b], sem).wait()

    for sl in range(nslab):
      pltpu.sync_copy(src_hbm.at[wid, pl.ds(sl * sch, sch)], idx_sl.at[0])
      pltpu.sync_copy(dst_hbm.at[wid, pl.ds(sl * sch, sch)], idx_sl.at[1])
      for b in range(3):
        _gather(b, b, gsem[b])

      def _quad(i, carry):
        for u in range(4):
          _wait(gsem[u], u)
          _scatter(4 * i + u, u, ssem[u])
          if u == 0:
            @pl.when(i > 0)
            def _():
              _wait(ssem[3], 3)
            _gather(4 * i + 3, 3, gsem[3])
          else:
            _wait(ssem[u - 1], u - 1)

            @pl.when(i < m - 1)
            def _():
              _gather(4 * i + u + 3, u - 1, gsem[u - 1])
        return carry

      lax.fori_loop(0, m, _quad, 0)
      _wait(ssem[3], 3)
    plsc.subcore_barrier()

    for k in range(_KPT):
      r0 = (s * _KPT + k) * _ZCH
      pltpu.sync_copy(acc_sh.at[pl.ds(r0, _ZCH)], zbuf_v)
      pltpu.sync_copy(zbuf_v, out_hbm.at[c, pl.ds(r0, _ZCH)])

  return agg


_agg_x = _make_edge_agg(_F, 64, 40)
_agg_q = _make_edge_agg(_CP, 128, 16, tc_tiling=False)


def _tc_transform(p, w1, w2p):
  def body(p0, p1, a, b, o):
    t = jnp.dot(p0[...] + p1[...], a[...], preferred_element_type=jnp.float32)
    t = jnp.maximum(t, 0.0)
    o[...] = jnp.dot(t, b[...], preferred_element_type=jnp.float32)

  blk = 1280
  return pl.pallas_call(
      body,
      grid=(_NPAD // blk,),
      in_specs=[
          pl.BlockSpec((None, blk, _F), lambda i: (0, i, 0)),
          pl.BlockSpec((None, blk, _F), lambda i: (1, i, 0)),
          pl.BlockSpec((_F, _F), lambda i: (0, 0)),
          pl.BlockSpec((_F, _CP), lambda i: (0, 0)),
      ],
      out_specs=pl.BlockSpec((blk, _CP), lambda i: (i, 0)),
      out_shape=jax.ShapeDtypeStruct((_NPAD, _CP), jnp.float32),
  )(p, p, w1, w2p)


def _tc_pool(p2, batch2d):
  def body(p0, p1, b, o):
    h2 = p0[...] + p1[...]
    gids = lax.broadcasted_iota(jnp.int32, (_NPAD, _G), 1)
    onehot = jnp.where(b[...] == gids, 1.0, 0.0).astype(jnp.float32)
    sums = lax.dot_general(onehot, h2, (((0,), (0,)), ((), ())),
                           preferred_element_type=jnp.float32)
    counts = jnp.maximum(jnp.sum(onehot, axis=0), 1.0)
    o[...] = sums / counts[:, None]

  return pl.pallas_call(
      body,
      grid=(1,),
      in_specs=[
          pl.BlockSpec((None, _NPAD, _CP), lambda i: (0, 0, 0)),
          pl.BlockSpec((None, _NPAD, _CP), lambda i: (1, 0, 0)),
          pl.BlockSpec((_NPAD, 1), lambda i: (0, 0)),
      ],
      out_specs=pl.BlockSpec((_G, _CP), lambda i: (0, 0)),
      out_shape=jax.ShapeDtypeStruct((_G, _CP), jnp.float32),
  )(p2, p2, batch2d)


def kernel(x, edge_index, batch, W1, W2):
  src = edge_index[0].astype(jnp.int32)
  dst = edge_index[1].astype(jnp.int32)
  pad = _EPAD - _E
  ar = jnp.arange(pad, dtype=jnp.int32)
  src_p = jnp.concatenate([src, ar % _N])
  dst_p = jnp.concatenate([dst, _N + (ar % (_NPAD - _N))])

  p1 = _agg_x(x, src_p.reshape(_NW, _EPW // 64, 64),
              dst_p.reshape(_NW, _EPW // 64, 64))
  w2p = jnp.pad(W2, ((0, 0), (0, _CP - _C)))
  q = _tc_transform(p1, W1, w2p)
  p2 = _agg_q(q, src_p.reshape(_NW, _EPW // 128, 128),
              dst_p.reshape(_NW, _EPW // 128, 128))

  bpad = jnp.concatenate(
      [batch.astype(jnp.int32), jnp.full((_NPAD - _N,), _G, jnp.int32)])
  out = _tc_pool(p2, bpad.reshape(_NPAD, 1))
  return out[:, :_C]
